# baseline (speedup 1.0000x reference)
import jax
import jax.numpy as jnp
from jax.scipy.linalg import block_diag
from jax.experimental import pallas as pl

B, S, C, D = 1024, 200, 32, 32
NIH, NAH, NUH = 2, 2, 4

SCH = 8
CCH = 8
HG = S // SCH
CG = C // CCH
GRID = HG + CG


def _fused_kernel(xp_ref, xa_ref, xact_ref, xprod_ref,
                  xcp_ref, xca_ref, xcs_ref,
                  xu_ref, u0_ref, h0_ref, c0_ref,
                  bd3p_ref, bd3a_ref, bd3act_ref, bd3s_ref,
                  bd2p_ref, bd2a_ref, bd2s_ref, wt1_ref,
                  hist_ref, cand_ref, uout_ref, umask_ref, hmask_ref, cmask_ref):
    f32 = jnp.float32
    i = pl.program_id(0)

    @pl.when(i < HG)
    def _history():
        for j in range(SCH * D // 128):
            acc = jnp.dot(bd3p_ref[...], xp_ref[j * 256:(j + 1) * 256, :],
                          preferred_element_type=f32)
            acc += jnp.dot(bd3a_ref[...], xa_ref[j * 256:(j + 1) * 256, :],
                           preferred_element_type=f32)
            acc += jnp.dot(bd3act_ref[...], xact_ref[j * 128:(j + 1) * 128, :],
                           preferred_element_type=f32)
            acc += jnp.dot(bd3s_ref[...], xprod_ref[j * 128:(j + 1) * 128, :],
                           preferred_element_type=f32)
            hist_ref[j * 128:(j + 1) * 128, :] = acc

    @pl.when(i >= HG)
    def _candidate():
        for j in range(CCH * D // 128):
            acc = jnp.dot(bd2p_ref[...], xcp_ref[j * 256:(j + 1) * 256, :],
                          preferred_element_type=f32)
            acc += jnp.dot(bd2a_ref[...], xca_ref[j * 256:(j + 1) * 256, :],
                           preferred_element_type=f32)
            acc += jnp.dot(bd2s_ref[...], xcs_ref[j * 128:(j + 1) * 128, :],
                           preferred_element_type=f32)
            cand_ref[j * 128:(j + 1) * 128, :] = acc

    @pl.when(i == GRID - 1)
    def _user_and_masks():
        uout_ref[...] = jnp.dot(wt1_ref[...], xu_ref[...],
                                preferred_element_type=f32)
        umask_ref[...] = u0_ref[0:1, :] != 0
        hmask_ref[...] = h0_ref[:, 0, :] != 0
        cmask_ref[...] = c0_ref[:, 0, :] != 0


def _bd4(w):
    return block_diag(w, w, w, w)


def kernel(user_hashes, user_embeddings, history_post_hashes, history_post_embeddings,
           history_author_embeddings, history_product_surface_embeddings,
           history_actions_embeddings, candidate_post_hashes, candidate_post_embeddings,
           candidate_author_embeddings, candidate_product_surface_embeddings,
           proj_mat_1, proj_mat_2, proj_mat_3):
    f32 = jnp.float32
    xp = history_post_embeddings.transpose(1, 2, 3, 0).reshape(S * NIH * D, B)
    xa = history_author_embeddings.transpose(1, 2, 3, 0).reshape(S * NAH * D, B)
    xact = history_actions_embeddings.transpose(1, 2, 0).reshape(S * D, B)
    xprod = history_product_surface_embeddings.transpose(1, 2, 0).reshape(S * D, B)
    xcp = candidate_post_embeddings.transpose(1, 2, 3, 0).reshape(C * NIH * D, B)
    xca = candidate_author_embeddings.transpose(1, 2, 3, 0).reshape(C * NAH * D, B)
    xcs = candidate_product_surface_embeddings.transpose(1, 2, 0).reshape(C * D, B)
    xu = user_embeddings.transpose(1, 2, 0).reshape(NUH * D, B)
    u0 = user_hashes.transpose(1, 0).astype(jnp.int32)
    h0 = history_post_hashes.transpose(1, 2, 0).astype(jnp.int32)
    c0 = candidate_post_hashes.transpose(1, 2, 0).astype(jnp.int32)

    wt1 = proj_mat_1.T
    wt2, wt3 = proj_mat_2.T, proj_mat_3.T
    bd2p = _bd4(wt2[:, : NIH * D])
    bd2a = _bd4(wt2[:, NIH * D:(NIH + NAH) * D])
    bd2s = _bd4(wt2[:, (NIH + NAH) * D:])
    bd3p = _bd4(wt3[:, : NIH * D])
    bd3a = _bd4(wt3[:, NIH * D:(NIH + NAH) * D])
    bd3act = _bd4(wt3[:, (NIH + NAH) * D:(NIH + NAH + 1) * D])
    bd3s = _bd4(wt3[:, (NIH + NAH + 1) * D:])

    hidx = lambda i: jnp.minimum(i, HG - 1)
    cidx = lambda i: jnp.clip(i - HG, 0, CG - 1)
    hspec = lambda r: pl.BlockSpec((r, B), lambda i: (hidx(i), 0))
    cspec = lambda r: pl.BlockSpec((r, B), lambda i: (cidx(i), 0))
    wspec = lambda r, c: pl.BlockSpec((r, c), lambda i: (0, 0))

    out_shapes = (
        jax.ShapeDtypeStruct((S * D, B), f32),
        jax.ShapeDtypeStruct((C * D, B), f32),
        jax.ShapeDtypeStruct((D, B), f32),
        jax.ShapeDtypeStruct((1, B), jnp.bool_),
        jax.ShapeDtypeStruct((S, B), jnp.bool_),
        jax.ShapeDtypeStruct((C, B), jnp.bool_),
    )
    out_specs = (
        hspec(SCH * D),
        cspec(CCH * D),
        wspec(D, B), wspec(1, B), wspec(S, B), wspec(C, B),
    )
    in_specs = (
        hspec(SCH * NIH * D), hspec(SCH * NAH * D), hspec(SCH * D), hspec(SCH * D),
        cspec(CCH * NIH * D), cspec(CCH * NAH * D), cspec(CCH * D),
        wspec(NUH * D, B), wspec(NUH, B),
        pl.BlockSpec((S, NIH, B), lambda i: (0, 0, 0)),
        pl.BlockSpec((C, NIH, B), lambda i: (0, 0, 0)),
        wspec(128, 256), wspec(128, 256), wspec(128, 128), wspec(128, 128),
        wspec(128, 256), wspec(128, 256), wspec(128, 128),
        wspec(D, NUH * D),
    )

    hist, cand, user, umask, hmask, cmask = pl.pallas_call(
        _fused_kernel,
        grid=(GRID,),
        in_specs=in_specs,
        out_specs=out_specs,
        out_shape=out_shapes,
    )(xp, xa, xact, xprod, xcp, xca, xcs, xu, u0, h0, c0,
      bd3p, bd3a, bd3act, bd3s, bd2p, bd2a, bd2s, wt1)

    return (cand.reshape(C, D, B).transpose(2, 0, 1),
            cmask.transpose(1, 0),
            hist.reshape(S, D, B).transpose(2, 0, 1),
            hmask.transpose(1, 0),
            user.transpose(1, 0).reshape(B, 1, D),
            umask.transpose(1, 0))


# speedup vs baseline: 1.0131x; 1.0131x over previous
import jax
import jax.numpy as jnp
from jax.scipy.linalg import block_diag
from jax.experimental import pallas as pl

B, S, C, D = 1024, 200, 32, 32
NIH, NAH, NUH = 2, 2, 4

SCH = 20
CCH = 8
HG = S // SCH
CG = C // CCH
GRID = HG + CG


def _fused_kernel(xp_ref, xa_ref, xact_ref, xprod_ref,
                  xcp_ref, xca_ref, xcs_ref,
                  xu_ref, u0_ref, h0_ref, c0_ref,
                  bd3p_ref, bd3a_ref, bd3act_ref, bd3s_ref,
                  bd2p_ref, bd2a_ref, bd2s_ref, wt1_ref,
                  hist_ref, cand_ref, uout_ref, umask_ref, hmask_ref, cmask_ref):
    f32 = jnp.float32
    i = pl.program_id(0)

    @pl.when(i < HG)
    def _history():
        for j in range(SCH * D // 128):
            acc = jnp.dot(bd3p_ref[...], xp_ref[j * 256:(j + 1) * 256, :],
                          preferred_element_type=f32)
            acc += jnp.dot(bd3a_ref[...], xa_ref[j * 256:(j + 1) * 256, :],
                           preferred_element_type=f32)
            acc += jnp.dot(bd3act_ref[...], xact_ref[j * 128:(j + 1) * 128, :],
                           preferred_element_type=f32)
            acc += jnp.dot(bd3s_ref[...], xprod_ref[j * 128:(j + 1) * 128, :],
                           preferred_element_type=f32)
            hist_ref[j * 128:(j + 1) * 128, :] = acc

    @pl.when(i >= HG)
    def _candidate():
        for j in range(CCH * D // 128):
            acc = jnp.dot(bd2p_ref[...], xcp_ref[j * 256:(j + 1) * 256, :],
                          preferred_element_type=f32)
            acc += jnp.dot(bd2a_ref[...], xca_ref[j * 256:(j + 1) * 256, :],
                           preferred_element_type=f32)
            acc += jnp.dot(bd2s_ref[...], xcs_ref[j * 128:(j + 1) * 128, :],
                           preferred_element_type=f32)
            cand_ref[j * 128:(j + 1) * 128, :] = acc

    @pl.when(i == GRID - 1)
    def _user_and_masks():
        uout_ref[...] = jnp.dot(wt1_ref[...], xu_ref[...],
                                preferred_element_type=f32)
        umask_ref[...] = u0_ref[0:1, :] != 0
        hmask_ref[...] = h0_ref[:, 0, :] != 0
        cmask_ref[...] = c0_ref[:, 0, :] != 0


def _bd4(w):
    return block_diag(w, w, w, w)


def kernel(user_hashes, user_embeddings, history_post_hashes, history_post_embeddings,
           history_author_embeddings, history_product_surface_embeddings,
           history_actions_embeddings, candidate_post_hashes, candidate_post_embeddings,
           candidate_author_embeddings, candidate_product_surface_embeddings,
           proj_mat_1, proj_mat_2, proj_mat_3):
    f32 = jnp.float32
    xp = history_post_embeddings.transpose(1, 2, 3, 0).reshape(S * NIH * D, B)
    xa = history_author_embeddings.transpose(1, 2, 3, 0).reshape(S * NAH * D, B)
    xact = history_actions_embeddings.transpose(1, 2, 0).reshape(S * D, B)
    xprod = history_product_surface_embeddings.transpose(1, 2, 0).reshape(S * D, B)
    xcp = candidate_post_embeddings.transpose(1, 2, 3, 0).reshape(C * NIH * D, B)
    xca = candidate_author_embeddings.transpose(1, 2, 3, 0).reshape(C * NAH * D, B)
    xcs = candidate_product_surface_embeddings.transpose(1, 2, 0).reshape(C * D, B)
    xu = user_embeddings.transpose(1, 2, 0).reshape(NUH * D, B)
    u0 = user_hashes.transpose(1, 0).astype(jnp.int32)
    h0 = history_post_hashes.transpose(1, 2, 0).astype(jnp.int32)
    c0 = candidate_post_hashes.transpose(1, 2, 0).astype(jnp.int32)

    wt1 = proj_mat_1.T
    wt2, wt3 = proj_mat_2.T, proj_mat_3.T
    bd2p = _bd4(wt2[:, : NIH * D])
    bd2a = _bd4(wt2[:, NIH * D:(NIH + NAH) * D])
    bd2s = _bd4(wt2[:, (NIH + NAH) * D:])
    bd3p = _bd4(wt3[:, : NIH * D])
    bd3a = _bd4(wt3[:, NIH * D:(NIH + NAH) * D])
    bd3act = _bd4(wt3[:, (NIH + NAH) * D:(NIH + NAH + 1) * D])
    bd3s = _bd4(wt3[:, (NIH + NAH + 1) * D:])

    hidx = lambda i: jnp.minimum(i, HG - 1)
    cidx = lambda i: jnp.clip(i - HG, 0, CG - 1)
    hspec = lambda r: pl.BlockSpec((r, B), lambda i: (hidx(i), 0))
    cspec = lambda r: pl.BlockSpec((r, B), lambda i: (cidx(i), 0))
    wspec = lambda r, c: pl.BlockSpec((r, c), lambda i: (0, 0))

    out_shapes = (
        jax.ShapeDtypeStruct((S * D, B), f32),
        jax.ShapeDtypeStruct((C * D, B), f32),
        jax.ShapeDtypeStruct((D, B), f32),
        jax.ShapeDtypeStruct((1, B), jnp.bool_),
        jax.ShapeDtypeStruct((S, B), jnp.bool_),
        jax.ShapeDtypeStruct((C, B), jnp.bool_),
    )
    out_specs = (
        hspec(SCH * D),
        cspec(CCH * D),
        wspec(D, B), wspec(1, B), wspec(S, B), wspec(C, B),
    )
    in_specs = (
        hspec(SCH * NIH * D), hspec(SCH * NAH * D), hspec(SCH * D), hspec(SCH * D),
        cspec(CCH * NIH * D), cspec(CCH * NAH * D), cspec(CCH * D),
        wspec(NUH * D, B), wspec(NUH, B),
        pl.BlockSpec((S, NIH, B), lambda i: (0, 0, 0)),
        pl.BlockSpec((C, NIH, B), lambda i: (0, 0, 0)),
        wspec(128, 256), wspec(128, 256), wspec(128, 128), wspec(128, 128),
        wspec(128, 256), wspec(128, 256), wspec(128, 128),
        wspec(D, NUH * D),
    )

    hist, cand, user, umask, hmask, cmask = pl.pallas_call(
        _fused_kernel,
        grid=(GRID,),
        in_specs=in_specs,
        out_specs=out_specs,
        out_shape=out_shapes,
    )(xp, xa, xact, xprod, xcp, xca, xcs, xu, u0, h0, c0,
      bd3p, bd3a, bd3act, bd3s, bd2p, bd2a, bd2s, wt1)

    return (cand.reshape(C, D, B).transpose(2, 0, 1),
            cmask.transpose(1, 0),
            hist.reshape(S, D, B).transpose(2, 0, 1),
            hmask.transpose(1, 0),
            user.transpose(1, 0).reshape(B, 1, D),
            umask.transpose(1, 0))


# speedup vs baseline: 1.1063x; 1.0920x over previous
import jax
import jax.numpy as jnp
from jax.experimental import pallas as pl
from jax.experimental.pallas import tpu as pltpu

B, S, C, D = 1024, 200, 32, 32
NIH, NAH, NUH = 2, 2, 4

SCH = 20
CCH = 8
HG = S // SCH
CG = C // CCH
GRID = HG + CG


def _fill_bd4(bd_ref, wt, k):
    bd_ref[...] = jnp.zeros_like(bd_ref)
    for i in range(4):
        bd_ref[32 * i:32 * (i + 1), k * i:k * (i + 1)] = wt


def _fused_kernel(xp_ref, xa_ref, xact_ref, xprod_ref,
                  xcp_ref, xca_ref, xcs_ref,
                  xu_ref, u0_ref, h0_ref, c0_ref,
                  wt1_ref, wt2_ref, wt3_ref,
                  hist_ref, cand_ref, uout_ref, umask_ref, hmask_ref, cmask_ref,
                  bd3p, bd3a, bd3act, bd3s, bd2p, bd2a, bd2s):
    f32 = jnp.float32
    i = pl.program_id(0)

    @pl.when(i == 0)
    def _build_weights():
        wt2 = wt2_ref[...]
        wt3 = wt3_ref[...]
        _fill_bd4(bd3p, wt3[:, : NIH * D], NIH * D)
        _fill_bd4(bd3a, wt3[:, NIH * D:(NIH + NAH) * D], NAH * D)
        _fill_bd4(bd3act, wt3[:, (NIH + NAH) * D:(NIH + NAH + 1) * D], D)
        _fill_bd4(bd3s, wt3[:, (NIH + NAH + 1) * D:], D)
        _fill_bd4(bd2p, wt2[:, : NIH * D], NIH * D)
        _fill_bd4(bd2a, wt2[:, NIH * D:(NIH + NAH) * D], NAH * D)
        _fill_bd4(bd2s, wt2[:, (NIH + NAH) * D:], D)

    @pl.when(i < HG)
    def _history():
        for j in range(SCH * D // 128):
            acc = jnp.dot(bd3p[...], xp_ref[j * 256:(j + 1) * 256, :],
                          preferred_element_type=f32)
            acc += jnp.dot(bd3a[...], xa_ref[j * 256:(j + 1) * 256, :],
                           preferred_element_type=f32)
            acc += jnp.dot(bd3act[...], xact_ref[j * 128:(j + 1) * 128, :],
                           preferred_element_type=f32)
            acc += jnp.dot(bd3s[...], xprod_ref[j * 128:(j + 1) * 128, :],
                           preferred_element_type=f32)
            hist_ref[j * 128:(j + 1) * 128, :] = acc

    @pl.when(i >= HG)
    def _candidate():
        for j in range(CCH * D // 128):
            acc = jnp.dot(bd2p[...], xcp_ref[j * 256:(j + 1) * 256, :],
                          preferred_element_type=f32)
            acc += jnp.dot(bd2a[...], xca_ref[j * 256:(j + 1) * 256, :],
                           preferred_element_type=f32)
            acc += jnp.dot(bd2s[...], xcs_ref[j * 128:(j + 1) * 128, :],
                           preferred_element_type=f32)
            cand_ref[j * 128:(j + 1) * 128, :] = acc

    @pl.when(i == GRID - 1)
    def _user_and_masks():
        uout_ref[...] = jnp.dot(wt1_ref[...], xu_ref[...],
                                preferred_element_type=f32)
        umask_ref[...] = u0_ref[0:1, :] != 0
        hmask_ref[...] = h0_ref[:, 0, :] != 0
        cmask_ref[...] = c0_ref[:, 0, :] != 0


def kernel(user_hashes, user_embeddings, history_post_hashes, history_post_embeddings,
           history_author_embeddings, history_product_surface_embeddings,
           history_actions_embeddings, candidate_post_hashes, candidate_post_embeddings,
           candidate_author_embeddings, candidate_product_surface_embeddings,
           proj_mat_1, proj_mat_2, proj_mat_3):
    f32 = jnp.float32
    xp = history_post_embeddings.transpose(1, 2, 3, 0).reshape(S * NIH * D, B)
    xa = history_author_embeddings.transpose(1, 2, 3, 0).reshape(S * NAH * D, B)
    xact = history_actions_embeddings.transpose(1, 2, 0).reshape(S * D, B)
    xprod = history_product_surface_embeddings.transpose(1, 2, 0).reshape(S * D, B)
    xcp = candidate_post_embeddings.transpose(1, 2, 3, 0).reshape(C * NIH * D, B)
    xca = candidate_author_embeddings.transpose(1, 2, 3, 0).reshape(C * NAH * D, B)
    xcs = candidate_product_surface_embeddings.transpose(1, 2, 0).reshape(C * D, B)
    xu = user_embeddings.transpose(1, 2, 0).reshape(NUH * D, B)
    u0 = user_hashes.transpose(1, 0).astype(jnp.int32)
    h0 = history_post_hashes.transpose(1, 2, 0).astype(jnp.int32)
    c0 = candidate_post_hashes.transpose(1, 2, 0).astype(jnp.int32)
    wt1 = proj_mat_1.T
    wt2, wt3 = proj_mat_2.T, proj_mat_3.T

    hidx = lambda i: jnp.minimum(i, HG - 1)
    cidx = lambda i: jnp.clip(i - HG, 0, CG - 1)
    hspec = lambda r: pl.BlockSpec((r, B), lambda i: (hidx(i), 0))
    cspec = lambda r: pl.BlockSpec((r, B), lambda i: (cidx(i), 0))
    wspec = lambda r, c: pl.BlockSpec((r, c), lambda i: (0, 0))

    out_shapes = (
        jax.ShapeDtypeStruct((S * D, B), f32),
        jax.ShapeDtypeStruct((C * D, B), f32),
        jax.ShapeDtypeStruct((D, B), f32),
        jax.ShapeDtypeStruct((1, B), jnp.bool_),
        jax.ShapeDtypeStruct((S, B), jnp.bool_),
        jax.ShapeDtypeStruct((C, B), jnp.bool_),
    )
    out_specs = (
        hspec(SCH * D),
        cspec(CCH * D),
        wspec(D, B), wspec(1, B), wspec(S, B), wspec(C, B),
    )
    in_specs = (
        hspec(SCH * NIH * D), hspec(SCH * NAH * D), hspec(SCH * D), hspec(SCH * D),
        cspec(CCH * NIH * D), cspec(CCH * NAH * D), cspec(CCH * D),
        wspec(NUH * D, B), wspec(NUH, B),
        pl.BlockSpec((S, NIH, B), lambda i: (0, 0, 0)),
        pl.BlockSpec((C, NIH, B), lambda i: (0, 0, 0)),
        wspec(D, NUH * D), wspec(D, (NIH + NAH + 1) * D), wspec(D, (NIH + NAH + 2) * D),
    )
    scratch_shapes = [
        pltpu.VMEM((128, 256), f32), pltpu.VMEM((128, 256), f32),
        pltpu.VMEM((128, 128), f32), pltpu.VMEM((128, 128), f32),
        pltpu.VMEM((128, 256), f32), pltpu.VMEM((128, 256), f32),
        pltpu.VMEM((128, 128), f32),
    ]

    hist, cand, user, umask, hmask, cmask = pl.pallas_call(
        _fused_kernel,
        grid=(GRID,),
        in_specs=in_specs,
        out_specs=out_specs,
        out_shape=out_shapes,
        scratch_shapes=scratch_shapes,
    )(xp, xa, xact, xprod, xcp, xca, xcs, xu, u0, h0, c0, wt1, wt2, wt3)

    return (cand.reshape(C, D, B).transpose(2, 0, 1),
            cmask.transpose(1, 0),
            hist.reshape(S, D, B).transpose(2, 0, 1),
            hmask.transpose(1, 0),
            user.transpose(1, 0).reshape(B, 1, D),
            umask.transpose(1, 0))
